# baseline (speedup 1.0000x reference)
import functools

import jax
import jax.numpy as jnp
from jax import lax
from jax.experimental import pallas as pl
from jax.experimental.pallas import tpu as pltpu
from jax.experimental.pallas import tpu_sc as plsc

_EPS = 1e-09
_TPAD = 1024

_D = 8
_RR = 512


def _coef_rows(betas_ref, t_ref):
    t = t_ref[...]
    b = t.shape[1]
    acc_le = jnp.zeros((1, b), jnp.float32)
    acc_eql = jnp.zeros((1, b), jnp.float32)
    acc_eqb = jnp.zeros((1, b), jnp.float32)
    ck = 256
    for k in range(_TPAD // ck):
        beta_c = betas_ref[k * ck:(k + 1) * ck, 0:1]
        la_c = jnp.log(1.0 - beta_c)
        jg = k * ck + lax.broadcasted_iota(jnp.int32, (ck, b), 0)
        le = jg <= t
        eq = jg == t
        acc_le += jnp.sum(jnp.where(le, la_c, 0.0), axis=0, keepdims=True)
        acc_eql += jnp.sum(jnp.where(eq, la_c, 0.0), axis=0, keepdims=True)
        acc_eqb += jnp.sum(jnp.where(eq, beta_c, 0.0), axis=0, keepdims=True)

    ac = jnp.exp(acc_le)
    acp = jnp.exp(acc_le - acc_eql)
    beta_t = acc_eqb
    alpha_t = 1.0 - beta_t
    recip = 1.0 / ac
    a1 = jnp.sqrt(recip)
    a2 = jnp.sqrt(recip - 1.0)
    om_ac = 1.0 - ac
    pvm = (1.0 - acp) / om_ac
    c1 = beta_t * jnp.sqrt(ac) / om_ac
    c2 = jnp.sqrt(alpha_t) * pvm
    return a1, a2, c1, c2


def _table_rows(betas_ref, tab_ref):
    ck = 256
    for ki in range(_TPAD // ck):
        lac = jnp.zeros((1, ck), jnp.float32)
        la_eq = jnp.zeros((1, ck), jnp.float32)
        beta_eq = jnp.zeros((1, ck), jnp.float32)
        ig = ki * ck + lax.broadcasted_iota(jnp.int32, (ck, ck), 1)
        for kj in range(ki + 1):
            beta_c = betas_ref[kj * ck:(kj + 1) * ck, 0:1]
            la_c = jnp.log(1.0 - beta_c)
            if kj < ki:
                lac += jnp.sum(la_c)
            else:
                jg = kj * ck + lax.broadcasted_iota(jnp.int32, (ck, ck), 0)
                le = jg <= ig
                eq = jg == ig
                lac += jnp.sum(jnp.where(le, la_c, 0.0), axis=0, keepdims=True)
                la_eq += jnp.sum(jnp.where(eq, la_c, 0.0), axis=0,
                                 keepdims=True)
                beta_eq += jnp.sum(jnp.where(eq, beta_c, 0.0), axis=0,
                                   keepdims=True)
        ac = jnp.exp(lac)
        acp = jnp.exp(lac - la_eq)
        pvm = (1.0 - acp) / (1.0 - ac)
        pv_t = beta_eq * pvm
        plv_t = jnp.log(jnp.maximum(pv_t, _EPS))
        tab_ref[ki:ki + 1, :] = pv_t
        tab_ref[4 + ki:5 + ki, :] = plv_t


def _fused_body(betas_ref, t_ref, x_hbm, n_hbm,
                xs_hbm, pm_hbm, tab_ref,
                xb, nb, xsb, pmb, six, sin, sox, sop):
    D, RR, B = xb.shape
    G = xs_hbm.shape[0] // RR
    KO = G // D
    assert KO * D == G, (G, D)

    def in_x(g, d):
        return pltpu.make_async_copy(
            x_hbm.at[pl.ds(g * RR, RR), :], xb.at[d], six.at[d])

    def in_n(g, d):
        return pltpu.make_async_copy(
            n_hbm.at[pl.ds(g * RR, RR), :], nb.at[d], sin.at[d])

    def out_xs(g, d):
        return pltpu.make_async_copy(
            xsb.at[d], xs_hbm.at[pl.ds(g * RR, RR), :], sox.at[d])

    def out_pm(g, d):
        return pltpu.make_async_copy(
            pmb.at[d], pm_hbm.at[pl.ds(g * RR, RR), :], sop.at[d])

    for d in range(D):
        in_x(d, d).start()
        in_n(d, d).start()

    a1, a2, c1, c2 = _coef_rows(betas_ref, t_ref)
    _table_rows(betas_ref, tab_ref)

    def outer(ko, carry):
        for d in range(D):
            g = ko * D + d
            in_x(g, d).wait()
            in_n(g, d).wait()

            @pl.when(ko > 0)
            def _():
                gp = (ko - 1) * D + d
                out_xs(gp, d).wait()
                out_pm(gp, d).wait()

            x = xb[d]
            n = nb[d]
            xs = a1 * x - a2 * n
            pm = c1 * xs + c2 * x
            xsb[d] = xs
            pmb[d] = pm
            out_xs(g, d).start()
            out_pm(g, d).start()

            @pl.when(ko < KO - 1)
            def _():
                gn = (ko + 1) * D + d
                in_x(gn, d).start()
                in_n(gn, d).start()
        return carry

    lax.fori_loop(0, KO, outer, 0)

    for d in range(D):
        gl = (KO - 1) * D + d
        out_xs(gl, d).wait()
        out_pm(gl, d).wait()


def _sc_gather_body(tab_hbm, t_hbm, pv_out, plv_out,
                    t_v, t2_v, pv_v, plv_v, sem):
    wid = lax.axis_index("c") * 16 + lax.axis_index("s")

    @pl.when(wid < 16)
    def _():
        base = wid * 16
        pltpu.sync_copy(t_hbm.at[pl.ds(base, 16)], t_v)
        t2_v[...] = t_v[...] + 1024
        pltpu.async_copy(tab_hbm.at[t_v], pv_v, sem).wait()
        pltpu.async_copy(tab_hbm.at[t2_v], plv_v, sem).wait()
        pltpu.sync_copy(pv_v, pv_out.at[pl.ds(base, 16)])
        pltpu.sync_copy(plv_v, plv_out.at[pl.ds(base, 16)])


def _sc_gather(tabf, t):
    mesh = plsc.VectorSubcoreMesh(core_axis_name="c", subcore_axis_name="s")
    B = t.shape[0]
    fn = functools.partial(
        pl.kernel, mesh=mesh,
        out_type=[jax.ShapeDtypeStruct((B,), jnp.float32),
                  jax.ShapeDtypeStruct((B,), jnp.float32)],
        scratch_types=[pltpu.VMEM((16,), jnp.int32),
                       pltpu.VMEM((16,), jnp.int32),
                       pltpu.VMEM((16,), jnp.float32),
                       pltpu.VMEM((16,), jnp.float32),
                       pltpu.SemaphoreType.DMA],
    )(_sc_gather_body)
    return fn(tabf, t)


def kernel(x_t, noise, betas, t):
    B, C, H, W = x_t.shape
    F = C * H * W
    x2 = jnp.transpose(x_t, (1, 2, 3, 0)).reshape(F, B)
    n2 = jnp.transpose(noise, (1, 2, 3, 0)).reshape(F, B)
    tlen = betas.shape[0]
    betas_col = jnp.concatenate(
        [betas, jnp.full((_TPAD - tlen,), 0.5, jnp.float32)]).reshape(_TPAD, 1)
    t_row = t.reshape(1, B)

    D, RR = _D, _RR
    xs2, pm2, tab = pl.pallas_call(
        _fused_body,
        in_specs=[
            pl.BlockSpec((_TPAD, 1), lambda: (0, 0)),
            pl.BlockSpec((1, B), lambda: (0, 0)),
            pl.BlockSpec(memory_space=pl.ANY),
            pl.BlockSpec(memory_space=pl.ANY),
        ],
        out_specs=[
            pl.BlockSpec(memory_space=pl.ANY),
            pl.BlockSpec(memory_space=pl.ANY),
            pl.BlockSpec((8, 256), lambda: (0, 0)),
        ],
        out_shape=[
            jax.ShapeDtypeStruct((F, B), jnp.float32),
            jax.ShapeDtypeStruct((F, B), jnp.float32),
            jax.ShapeDtypeStruct((8, 256), jnp.float32),
        ],
        scratch_shapes=[
            pltpu.VMEM((D, RR, B), jnp.float32),
            pltpu.VMEM((D, RR, B), jnp.float32),
            pltpu.VMEM((D, RR, B), jnp.float32),
            pltpu.VMEM((D, RR, B), jnp.float32),
            pltpu.SemaphoreType.DMA((D,)),
            pltpu.SemaphoreType.DMA((D,)),
            pltpu.SemaphoreType.DMA((D,)),
            pltpu.SemaphoreType.DMA((D,)),
        ],
    )(betas_col, t_row, x2, n2)

    tabf = tab.reshape(2 * _TPAD)
    pv, plv = _sc_gather(tabf, t)

    xs = jnp.transpose(xs2.reshape(C, H, W, B), (3, 0, 1, 2))
    pm = jnp.transpose(pm2.reshape(C, H, W, B), (3, 0, 1, 2))
    return (xs, pm, pv, plv)


# speedup vs baseline: 1.0306x; 1.0306x over previous
import functools

import jax
import jax.numpy as jnp
from jax import lax
from jax.experimental import pallas as pl
from jax.experimental.pallas import tpu as pltpu
from jax.experimental.pallas import tpu_sc as plsc

_EPS = 1e-09
_TPAD = 1024

_D = 8
_RR = 512


def _coef_rows(betas_ref, t_ref):
    t = t_ref[...]
    b = t.shape[1]
    acc_le = jnp.zeros((1, b), jnp.float32)
    acc_eql = jnp.zeros((1, b), jnp.float32)
    acc_eqb = jnp.zeros((1, b), jnp.float32)
    ck = 256
    for k in range(_TPAD // ck):
        beta_c = betas_ref[k * ck:(k + 1) * ck, 0:1]
        la_c = jnp.log(1.0 - beta_c)
        jg = k * ck + lax.broadcasted_iota(jnp.int32, (ck, b), 0)
        le = jg <= t
        eq = jg == t
        acc_le += jnp.sum(jnp.where(le, la_c, 0.0), axis=0, keepdims=True)
        acc_eql += jnp.sum(jnp.where(eq, la_c, 0.0), axis=0, keepdims=True)
        acc_eqb += jnp.sum(jnp.where(eq, beta_c, 0.0), axis=0, keepdims=True)

    ac = jnp.exp(acc_le)
    acp = jnp.exp(acc_le - acc_eql)
    beta_t = acc_eqb
    alpha_t = 1.0 - beta_t
    recip = 1.0 / ac
    a1 = jnp.sqrt(recip)
    a2 = jnp.sqrt(recip - 1.0)
    om_ac = 1.0 - ac
    pvm = (1.0 - acp) / om_ac
    c1 = beta_t * jnp.sqrt(ac) / om_ac
    c2 = jnp.sqrt(alpha_t) * pvm
    return a1, a2, c1, c2


def _table_rows(betas_ref, tab_ref):
    ck = 256
    for ki in range(_TPAD // ck):
        lac = jnp.zeros((1, ck), jnp.float32)
        la_eq = jnp.zeros((1, ck), jnp.float32)
        beta_eq = jnp.zeros((1, ck), jnp.float32)
        ig = ki * ck + lax.broadcasted_iota(jnp.int32, (ck, ck), 1)
        for kj in range(ki + 1):
            beta_c = betas_ref[kj * ck:(kj + 1) * ck, 0:1]
            la_c = jnp.log(1.0 - beta_c)
            if kj < ki:
                lac += jnp.sum(la_c)
            else:
                jg = kj * ck + lax.broadcasted_iota(jnp.int32, (ck, ck), 0)
                le = jg <= ig
                eq = jg == ig
                lac += jnp.sum(jnp.where(le, la_c, 0.0), axis=0, keepdims=True)
                la_eq += jnp.sum(jnp.where(eq, la_c, 0.0), axis=0,
                                 keepdims=True)
                beta_eq += jnp.sum(jnp.where(eq, beta_c, 0.0), axis=0,
                                   keepdims=True)
        ac = jnp.exp(lac)
        acp = jnp.exp(lac - la_eq)
        pvm = (1.0 - acp) / (1.0 - ac)
        pv_t = beta_eq * pvm
        plv_t = jnp.log(jnp.maximum(pv_t, _EPS))
        tab_ref[ki:ki + 1, :] = pv_t
        tab_ref[4 + ki:5 + ki, :] = plv_t


def _prep_body(betas_ref, t_ref, coef_ref, tab_ref):
    a1, a2, c1, c2 = _coef_rows(betas_ref, t_ref)
    coef_ref[0:1, :] = a1
    coef_ref[1:2, :] = a2
    coef_ref[2:3, :] = c1
    coef_ref[3:4, :] = c2
    coef_ref[4:8, :] = jnp.zeros((4, a1.shape[1]), jnp.float32)
    _table_rows(betas_ref, tab_ref)


def _dense_body(coef_ref, x_hbm, n_hbm,
                xs_hbm, pm_hbm,
                xb, nb, xsb, pmb, six, sin, sox, sop):
    D, RR, B = xb.shape
    G = xs_hbm.shape[0] // RR
    KO = G // D
    assert KO * D == G, (G, D)

    def in_x(g, d):
        return pltpu.make_async_copy(
            x_hbm.at[pl.ds(g * RR, RR), :], xb.at[d], six.at[d])

    def in_n(g, d):
        return pltpu.make_async_copy(
            n_hbm.at[pl.ds(g * RR, RR), :], nb.at[d], sin.at[d])

    def out_xs(g, d):
        return pltpu.make_async_copy(
            xsb.at[d], xs_hbm.at[pl.ds(g * RR, RR), :], sox.at[d])

    def out_pm(g, d):
        return pltpu.make_async_copy(
            pmb.at[d], pm_hbm.at[pl.ds(g * RR, RR), :], sop.at[d])

    for d in range(D):
        in_x(d, d).start()
        in_n(d, d).start()

    a1 = coef_ref[0:1, :]
    a2 = coef_ref[1:2, :]
    c1 = coef_ref[2:3, :]
    c2 = coef_ref[3:4, :]

    def outer(ko, carry):
        for d in range(D):
            g = ko * D + d
            in_x(g, d).wait()
            in_n(g, d).wait()

            @pl.when(ko > 0)
            def _():
                gp = (ko - 1) * D + d
                out_xs(gp, d).wait()
                out_pm(gp, d).wait()

            x = xb[d]
            n = nb[d]
            xs = a1 * x - a2 * n
            pm = c1 * xs + c2 * x
            xsb[d] = xs
            pmb[d] = pm
            out_xs(g, d).start()
            out_pm(g, d).start()

            @pl.when(ko < KO - 1)
            def _():
                gn = (ko + 1) * D + d
                in_x(gn, d).start()
                in_n(gn, d).start()
        return carry

    lax.fori_loop(0, KO, outer, 0)

    for d in range(D):
        gl = (KO - 1) * D + d
        out_xs(gl, d).wait()
        out_pm(gl, d).wait()


def _sc_gather_body(tab_hbm, t_hbm, pv_out, plv_out,
                    t_v, t2_v, pv_v, plv_v, sem):
    wid = lax.axis_index("c") * 16 + lax.axis_index("s")

    @pl.when(wid < 16)
    def _():
        base = wid * 16
        pltpu.sync_copy(t_hbm.at[pl.ds(base, 16)], t_v)
        t2_v[...] = t_v[...] + 1024
        pltpu.async_copy(tab_hbm.at[t_v], pv_v, sem).wait()
        pltpu.async_copy(tab_hbm.at[t2_v], plv_v, sem).wait()
        pltpu.sync_copy(pv_v, pv_out.at[pl.ds(base, 16)])
        pltpu.sync_copy(plv_v, plv_out.at[pl.ds(base, 16)])


def _sc_gather(tabf, t):
    mesh = plsc.VectorSubcoreMesh(core_axis_name="c", subcore_axis_name="s")
    B = t.shape[0]
    fn = functools.partial(
        pl.kernel, mesh=mesh,
        out_type=[jax.ShapeDtypeStruct((B,), jnp.float32),
                  jax.ShapeDtypeStruct((B,), jnp.float32)],
        scratch_types=[pltpu.VMEM((16,), jnp.int32),
                       pltpu.VMEM((16,), jnp.int32),
                       pltpu.VMEM((16,), jnp.float32),
                       pltpu.VMEM((16,), jnp.float32),
                       pltpu.SemaphoreType.DMA],
    )(_sc_gather_body)
    return fn(tabf, t)


def kernel(x_t, noise, betas, t):
    B, C, H, W = x_t.shape
    F = C * H * W
    x2 = jnp.transpose(x_t, (1, 2, 3, 0)).reshape(F, B)
    n2 = jnp.transpose(noise, (1, 2, 3, 0)).reshape(F, B)
    tlen = betas.shape[0]
    betas_col = jnp.concatenate(
        [betas, jnp.full((_TPAD - tlen,), 0.5, jnp.float32)]).reshape(_TPAD, 1)
    t_row = t.reshape(1, B)

    coef, tab = pl.pallas_call(
        _prep_body,
        out_shape=[
            jax.ShapeDtypeStruct((8, B), jnp.float32),
            jax.ShapeDtypeStruct((8, 256), jnp.float32),
        ],
    )(betas_col, t_row)

    tabf = tab.reshape(2 * _TPAD)
    pv, plv = _sc_gather(tabf, t)

    D, RR = _D, _RR
    xs2, pm2 = pl.pallas_call(
        _dense_body,
        in_specs=[
            pl.BlockSpec((8, B), lambda: (0, 0)),
            pl.BlockSpec(memory_space=pl.ANY),
            pl.BlockSpec(memory_space=pl.ANY),
        ],
        out_specs=[
            pl.BlockSpec(memory_space=pl.ANY),
            pl.BlockSpec(memory_space=pl.ANY),
        ],
        out_shape=[
            jax.ShapeDtypeStruct((F, B), jnp.float32),
            jax.ShapeDtypeStruct((F, B), jnp.float32),
        ],
        scratch_shapes=[
            pltpu.VMEM((D, RR, B), jnp.float32),
            pltpu.VMEM((D, RR, B), jnp.float32),
            pltpu.VMEM((D, RR, B), jnp.float32),
            pltpu.VMEM((D, RR, B), jnp.float32),
            pltpu.SemaphoreType.DMA((D,)),
            pltpu.SemaphoreType.DMA((D,)),
            pltpu.SemaphoreType.DMA((D,)),
            pltpu.SemaphoreType.DMA((D,)),
        ],
    )(coef, x2, n2)

    xs = jnp.transpose(xs2.reshape(C, H, W, B), (3, 0, 1, 2))
    pm = jnp.transpose(pm2.reshape(C, H, W, B), (3, 0, 1, 2))
    return (xs, pm, pv, plv)


# speedup vs baseline: 1.7587x; 1.7064x over previous
import jax
import jax.numpy as jnp
from jax import lax
from jax.experimental import pallas as pl
from jax.experimental.pallas import tpu as pltpu

_EPS = 1e-09
_TPAD = 1024

_D = 8
_RR = 512


def _coef_rows(betas_ref, t_ref):
    t = t_ref[...]
    b = t.shape[1]
    acc_le = jnp.zeros((1, b), jnp.float32)
    acc_eql = jnp.zeros((1, b), jnp.float32)
    acc_eqb = jnp.zeros((1, b), jnp.float32)
    ck = 256
    for k in range(_TPAD // ck):
        beta_c = betas_ref[k * ck:(k + 1) * ck, 0:1]
        la_c = jnp.log(1.0 - beta_c)
        jg = k * ck + lax.broadcasted_iota(jnp.int32, (ck, b), 0)
        le = jg <= t
        eq = jg == t
        acc_le += jnp.sum(jnp.where(le, la_c, 0.0), axis=0, keepdims=True)
        acc_eql += jnp.sum(jnp.where(eq, la_c, 0.0), axis=0, keepdims=True)
        acc_eqb += jnp.sum(jnp.where(eq, beta_c, 0.0), axis=0, keepdims=True)

    ac = jnp.exp(acc_le)
    acp = jnp.exp(acc_le - acc_eql)
    beta_t = acc_eqb
    alpha_t = 1.0 - beta_t
    recip = 1.0 / ac
    a1 = jnp.sqrt(recip)
    a2 = jnp.sqrt(recip - 1.0)
    om_ac = 1.0 - ac
    pvm = (1.0 - acp) / om_ac
    pv = beta_t * pvm
    plv = jnp.log(jnp.maximum(pv, _EPS))
    c1 = beta_t * jnp.sqrt(ac) / om_ac
    c2 = jnp.sqrt(alpha_t) * pvm
    return a1, a2, c1, c2, pv, plv


def _fused_body(betas_ref, t_ref, x_hbm, n_hbm,
                xs_hbm, pm_hbm, pv_ref, plv_ref,
                xb, nb, xsb, pmb, six, sin, sox, sop):
    D, RR, B = xb.shape
    G = xs_hbm.shape[0] // RR
    KO = G // D
    assert KO * D == G, (G, D)

    def in_x(g, d):
        return pltpu.make_async_copy(
            x_hbm.at[pl.ds(g * RR, RR), :], xb.at[d], six.at[d])

    def in_n(g, d):
        return pltpu.make_async_copy(
            n_hbm.at[pl.ds(g * RR, RR), :], nb.at[d], sin.at[d])

    def out_xs(g, d):
        return pltpu.make_async_copy(
            xsb.at[d], xs_hbm.at[pl.ds(g * RR, RR), :], sox.at[d])

    def out_pm(g, d):
        return pltpu.make_async_copy(
            pmb.at[d], pm_hbm.at[pl.ds(g * RR, RR), :], sop.at[d])

    for d in range(D):
        in_x(d, d).start()
        in_n(d, d).start()

    a1, a2, c1, c2, pv, plv = _coef_rows(betas_ref, t_ref)
    pv_ref[...] = pv
    plv_ref[...] = plv

    def outer(ko, carry):
        for d in range(D):
            g = ko * D + d
            in_x(g, d).wait()
            in_n(g, d).wait()

            @pl.when(ko > 0)
            def _():
                gp = (ko - 1) * D + d
                out_xs(gp, d).wait()
                out_pm(gp, d).wait()

            x = xb[d]
            n = nb[d]
            xs = a1 * x - a2 * n
            pm = c1 * xs + c2 * x
            xsb[d] = xs
            pmb[d] = pm
            out_xs(g, d).start()
            out_pm(g, d).start()

            @pl.when(ko < KO - 1)
            def _():
                gn = (ko + 1) * D + d
                in_x(gn, d).start()
                in_n(gn, d).start()
        return carry

    lax.fori_loop(0, KO, outer, 0)

    for d in range(D):
        gl = (KO - 1) * D + d
        out_xs(gl, d).wait()
        out_pm(gl, d).wait()


def kernel(x_t, noise, betas, t):
    B, C, H, W = x_t.shape
    F = C * H * W
    x2 = jnp.transpose(x_t, (1, 2, 3, 0)).reshape(F, B)
    n2 = jnp.transpose(noise, (1, 2, 3, 0)).reshape(F, B)
    tlen = betas.shape[0]
    betas_col = jnp.concatenate(
        [betas, jnp.full((_TPAD - tlen,), 0.5, jnp.float32)]).reshape(_TPAD, 1)
    t_row = t.reshape(1, B)

    D, RR = _D, _RR
    xs2, pm2, pv, plv = pl.pallas_call(
        _fused_body,
        in_specs=[
            pl.BlockSpec((_TPAD, 1), lambda: (0, 0)),
            pl.BlockSpec((1, B), lambda: (0, 0)),
            pl.BlockSpec(memory_space=pl.ANY),
            pl.BlockSpec(memory_space=pl.ANY),
        ],
        out_specs=[
            pl.BlockSpec(memory_space=pl.ANY),
            pl.BlockSpec(memory_space=pl.ANY),
            pl.BlockSpec((1, B), lambda: (0, 0)),
            pl.BlockSpec((1, B), lambda: (0, 0)),
        ],
        out_shape=[
            jax.ShapeDtypeStruct((F, B), jnp.float32),
            jax.ShapeDtypeStruct((F, B), jnp.float32),
            jax.ShapeDtypeStruct((1, B), jnp.float32),
            jax.ShapeDtypeStruct((1, B), jnp.float32),
        ],
        scratch_shapes=[
            pltpu.VMEM((D, RR, B), jnp.float32),
            pltpu.VMEM((D, RR, B), jnp.float32),
            pltpu.VMEM((D, RR, B), jnp.float32),
            pltpu.VMEM((D, RR, B), jnp.float32),
            pltpu.SemaphoreType.DMA((D,)),
            pltpu.SemaphoreType.DMA((D,)),
            pltpu.SemaphoreType.DMA((D,)),
            pltpu.SemaphoreType.DMA((D,)),
        ],
    )(betas_col, t_row, x2, n2)

    xs = jnp.transpose(xs2.reshape(C, H, W, B), (3, 0, 1, 2))
    pm = jnp.transpose(pm2.reshape(C, H, W, B), (3, 0, 1, 2))
    return (xs, pm, pv.reshape(B), plv.reshape(B))
